# Initial kernel scaffold; baseline (speedup 1.0000x reference)
import jax
import jax.numpy as jnp
from jax.experimental import pallas as pl


def kernel(inputs, embedding):
    raise NotImplementedError("write your pallas kernel here")


# speedup vs baseline: 1.0230x; 1.0230x over previous
import functools

import jax
import jax.numpy as jnp
from jax import lax
from jax.experimental import pallas as pl
from jax.experimental.pallas import tpu as pltpu
from jax.experimental.pallas import tpu_sc as plsc

_NC, _NS = 2, 16
_NW = _NC * _NS
_CHUNK = 128


def _gather_body(table, idx_hbm, out, idx_v, rows_v, sem):
    c = lax.axis_index("c")
    s = lax.axis_index("s")
    wid = s * _NC + c
    n_chunks = idx_v.shape[0]
    base = wid * (n_chunks * _CHUNK)
    pltpu.sync_copy(idx_hbm.at[wid], idx_v)

    def step(j, carry):
        pltpu.async_copy(table.at[idx_v.at[j]], rows_v, sem).wait()
        pltpu.sync_copy(rows_v, out.at[pl.ds(base + j * _CHUNK, _CHUNK)])
        return carry

    lax.fori_loop(0, n_chunks, step, 0)


def kernel(inputs, embedding):
    if inputs.shape[-1] == 1:
        inputs = jnp.squeeze(inputs, axis=-1)
    lead_shape = inputs.shape
    b = 1
    for d in lead_shape:
        b *= d
    dim = embedding.shape[1]
    n_chunks = b // (_NW * _CHUNK)
    idx = inputs.reshape(_NW, n_chunks, _CHUNK).astype(jnp.int32)
    mesh = plsc.VectorSubcoreMesh(core_axis_name="c", subcore_axis_name="s")
    out = pl.kernel(
        _gather_body,
        out_type=jax.ShapeDtypeStruct((b, dim), jnp.float32),
        mesh=mesh,
        compiler_params=pltpu.CompilerParams(use_tc_tiling_on_sc=False),
        scratch_types=[
            pltpu.VMEM((n_chunks, _CHUNK), jnp.int32),
            pltpu.VMEM((_CHUNK, dim), jnp.float32),
            pltpu.SemaphoreType.DMA,
        ],
    )(embedding, idx)
    return out.reshape(*lead_shape, dim)


# speedup vs baseline: 1.1091x; 1.0841x over previous
import functools

import jax
import jax.numpy as jnp
from jax import lax
from jax.experimental import pallas as pl
from jax.experimental.pallas import tpu as pltpu
from jax.experimental.pallas import tpu_sc as plsc

_NC, _NS = 2, 16
_NW = _NC * _NS
_CHUNK = 128
_K = 5
_NBUF = 4
_GROUP = _K * _CHUNK


def _gather_body(table, idx_hbm, out, idx_v, bufs, *sems):
    gsems, wsems = sems[:_NBUF], sems[_NBUF:]
    c = lax.axis_index("c")
    s = lax.axis_index("s")
    wid = s * _NC + c
    n_chunks = idx_v.shape[0]
    n_groups = n_chunks // _K
    base = wid * (n_chunks * _CHUNK)
    pltpu.sync_copy(idx_hbm.at[wid], idx_v)

    def fire_gathers(g, i):
        for kk in range(_K):
            j = g * _K + kk
            pltpu.async_copy(
                table.at[idx_v.at[j]],
                bufs.at[i].at[pl.ds(kk * _CHUNK, _CHUNK)],
                gsems[i],
            )

    for i in range(_NBUF):
        fire_gathers(i, i)

    def outer(go, carry):
        for i in range(_NBUF):
            g = go * _NBUF + i
            pltpu.make_async_copy(table.at[pl.ds(0, _GROUP)], bufs.at[i],
                                  gsems[i]).wait()
            pltpu.async_copy(bufs.at[i], out.at[pl.ds(base + g * _GROUP, _GROUP)],
                             wsems[i])
        for i in range(_NBUF):
            pltpu.make_async_copy(bufs.at[i], out.at[pl.ds(base, _GROUP)],
                                  wsems[i]).wait()
            gn = (go + 1) * _NBUF + i

            @pl.when(gn < n_groups)
            def _():
                fire_gathers(gn, i)

        return carry

    lax.fori_loop(0, n_groups // _NBUF, outer, 0)


def kernel(inputs, embedding):
    if inputs.shape[-1] == 1:
        inputs = jnp.squeeze(inputs, axis=-1)
    lead_shape = inputs.shape
    b = 1
    for d in lead_shape:
        b *= d
    dim = embedding.shape[1]
    n_chunks = b // (_NW * _CHUNK)
    idx = inputs.reshape(_NW, n_chunks, _CHUNK).astype(jnp.int32)
    mesh = plsc.VectorSubcoreMesh(core_axis_name="c", subcore_axis_name="s")
    out = pl.kernel(
        _gather_body,
        out_type=jax.ShapeDtypeStruct((b, dim), jnp.float32),
        mesh=mesh,
        compiler_params=pltpu.CompilerParams(use_tc_tiling_on_sc=False),
        scratch_types=[
            pltpu.VMEM((n_chunks, _CHUNK), jnp.int32),
            pltpu.VMEM((_NBUF, _GROUP, dim), jnp.float32),
        ] + [pltpu.SemaphoreType.DMA] * (2 * _NBUF),
    )(embedding, idx)
    return out.reshape(*lead_shape, dim)


# speedup vs baseline: 1.1891x; 1.0721x over previous
import jax
import jax.numpy as jnp
from jax import lax
from jax.experimental import pallas as pl
from jax.experimental.pallas import tpu as pltpu
from jax.experimental.pallas import tpu_sc as plsc

_NC, _NS = 2, 16
_NW = _NC * _NS
_L = 16


def _wid():
    return lax.axis_index("s") * _NC + lax.axis_index("c")


def _requad_body(emb_t, emb_tail, tbl_q, in_v, out_v, isa, isb, osa, osb):
    wid = _wid()
    rows16 = lax.iota(jnp.int32, _L)
    iss = (isa, isb)
    oss = (osa, osb)

    def fire_in(c, s, n_cols=128):
        pltpu.async_copy(emb_t.at[:, pl.ds(c * 128, n_cols)],
                         in_v.at[s, :, pl.ds(0, n_cols)], iss[s])

    def wait_in(s):
        pltpu.make_async_copy(emb_t.at[:, pl.ds(0, 128)], in_v.at[s],
                              iss[s]).wait()

    def transpose(s, n_q=32):
        for a in range(n_q):
            for b in range(4):
                col = jnp.full((_L,), 4 * a + b, jnp.int32)
                for h in range(2):
                    vals = plsc.load_gather(in_v.at[s],
                                            [rows16 + 16 * h, col])
                    out_v[s, a, pl.ds(32 * b + 16 * h, _L)] = vals

    def fire_out(c, s, n_q=32):
        pltpu.async_copy(out_v.at[s, pl.ds(0, n_q)],
                         tbl_q.at[pl.ds(c * 32, n_q)], oss[s])

    def wait_out(s):
        pltpu.make_async_copy(out_v.at[s], tbl_q.at[pl.ds(0, 32)],
                              oss[s]).wait()

    def blk(k):
        return k * _NW + wid

    fire_in(blk(0), 0)

    def pair(p, carry):
        fire_in(blk(2 * p + 1), 1)
        wait_in(0)

        @pl.when(p > 0)
        def _():
            wait_out(0)

        transpose(0)
        fire_out(blk(2 * p), 0)

        @pl.when(p < 121)
        def _():
            fire_in(blk(2 * p + 2), 0)

        wait_in(1)

        @pl.when(p > 0)
        def _():
            wait_out(1)

        transpose(1)
        fire_out(blk(2 * p + 1), 1)
        return carry

    lax.fori_loop(0, 122, pair, 0)
    wait_out(0)
    wait_out(1)

    @pl.when(wid < 4)
    def _():
        fire_in(7808 + wid, 0)
        wait_in(0)
        transpose(0)
        fire_out(7808 + wid, 0)
        wait_out(0)

    @pl.when(wid == 4)
    def _():
        pltpu.async_copy(emb_tail, tbl_q.at[pl.ds(7812 * 32, 16)], osa).wait()


def _gather_body(tbl_q, idx_t, out_t, idx_v, q_v, r_v, g_v, o_v,
                 isem, gsa, gsb, osa, osb):
    wid = _wid()
    rows16 = lax.iota(jnp.int32, _L)
    gss = (gsa, gsb)
    oss = (osa, osb)

    def prep(j, s):
        for b in range(8):
            v = idx_v[j, pl.ds(16 * b, _L)]
            q_v[s, pl.ds(16 * b, _L)] = v >> 2
            r_v[s, pl.ds(16 * b, _L)] = (v & 3) * 32

    def fire(s):
        pltpu.async_copy(tbl_q.at[q_v.at[s]], g_v.at[s], gss[s])

    def drain(s):
        pltpu.make_async_copy(tbl_q.at[pl.ds(0, 128)], g_v.at[s],
                              gss[s]).wait()

    def extract(s):
        for lb in range(8):
            base = r_v[s, pl.ds(16 * lb, _L)]
            rr = rows16 + 16 * lb
            for f in range(32):
                vals = plsc.load_gather(g_v.at[s], [rr, base + f])
                o_v[s, f, pl.ds(16 * lb, _L)] = vals

    def wb(j, c, s):
        pltpu.async_copy(o_v.at[s], out_t.at[j, :, pl.ds(c * 128, 128)],
                         oss[s])

    def wb_wait(s):
        pltpu.make_async_copy(o_v.at[s], out_t.at[0, :, pl.ds(0, 128)],
                              oss[s]).wait()

    def do_cblock(cc, carry):
        c = cc * _NW + wid
        pltpu.async_copy(idx_t.at[:, pl.ds(c * 128, 128)], idx_v, isem)
        pltpu.make_async_copy(idx_t.at[:, pl.ds(0, 128)], idx_v, isem).wait()
        prep(0, 0)
        fire(0)

        def pair(p, carry2):
            j0 = 2 * p
            prep(j0 + 1, 1)
            fire(1)
            drain(0)

            @pl.when((cc > 0) | (p > 0))
            def _():
                wb_wait(0)

            extract(0)
            wb(j0, c, 0)

            @pl.when(p < 24)
            def _():
                prep(j0 + 2, 0)
                fire(0)

            drain(1)

            @pl.when((cc > 0) | (p > 0))
            def _():
                wb_wait(1)

            extract(1)
            wb(j0 + 1, c, 1)
            return carry2

        lax.fori_loop(0, 25, pair, 0)
        return carry

    lax.fori_loop(0, 16384 // 128 // _NW, do_cblock, 0)
    wb_wait(0)
    wb_wait(1)


def kernel(inputs, embedding):
    if inputs.shape[-1] == 1:
        inputs = jnp.squeeze(inputs, axis=-1)
    n, m = inputs.shape
    dim = embedding.shape[1]

    emb_t = jnp.transpose(embedding)
    idx_t = jnp.transpose(inputs).astype(jnp.int32)
    n_tail = embedding.shape[0] % 512
    emb_tail = jnp.reshape(
        lax.slice(embedding, (embedding.shape[0] - n_tail, 0),
                  (embedding.shape[0], dim)), (n_tail * dim // 128, 128))
    mesh = plsc.VectorSubcoreMesh(core_axis_name="c", subcore_axis_name="s")
    params = pltpu.CompilerParams(use_tc_tiling_on_sc=True,
                                  needs_layout_passes=False)

    tbl_q = pl.kernel(
        _requad_body,
        out_type=jax.ShapeDtypeStruct((250000, 128), jnp.float32),
        mesh=mesh,
        compiler_params=params,
        scratch_types=[
            pltpu.VMEM((2, dim, 128), jnp.float32),
            pltpu.VMEM((2, 32, 128), jnp.float32),
            pltpu.SemaphoreType.DMA,
            pltpu.SemaphoreType.DMA,
            pltpu.SemaphoreType.DMA,
            pltpu.SemaphoreType.DMA,
        ],
    )(emb_t, emb_tail)

    out_t = pl.kernel(
        _gather_body,
        out_type=jax.ShapeDtypeStruct((m, dim, n), jnp.float32),
        mesh=mesh,
        compiler_params=params,
        scratch_types=[
            pltpu.VMEM((m, 128), jnp.int32),
            pltpu.VMEM((2, 128), jnp.int32),
            pltpu.VMEM((2, 128), jnp.int32),
            pltpu.VMEM((2, 128, 128), jnp.float32),
            pltpu.VMEM((2, dim, 128), jnp.float32),
            pltpu.SemaphoreType.DMA,
            pltpu.SemaphoreType.DMA,
            pltpu.SemaphoreType.DMA,
            pltpu.SemaphoreType.DMA,
            pltpu.SemaphoreType.DMA,
        ],
    )(tbl_q, idx_t)

    return jnp.transpose(out_t, (2, 0, 1))


# speedup vs baseline: 2.4197x; 2.0350x over previous
import jax
import jax.numpy as jnp
from jax import lax
from jax.experimental import pallas as pl
from jax.experimental.pallas import tpu as pltpu
from jax.experimental.pallas import tpu_sc as plsc

_NC, _NS = 2, 16
_NW = _NC * _NS
_L = 16


def _wid():
    return lax.axis_index("s") * _NC + lax.axis_index("c")


def _requad_body(emb_t, emb_tail, tbl_q, in_v, out_v, isa, isb, osa, osb):
    wid = _wid()
    rows16 = lax.iota(jnp.int32, _L)
    iss = (isa, isb)
    oss = (osa, osb)

    def fire_in(c, s, n_cols=128):
        pltpu.async_copy(emb_t.at[:, pl.ds(c * 128, n_cols)],
                         in_v.at[s, :, pl.ds(0, n_cols)], iss[s])

    def wait_in(s):
        pltpu.make_async_copy(emb_t.at[:, pl.ds(0, 128)], in_v.at[s],
                              iss[s]).wait()

    def transpose(s, n_q=32):
        @plsc.parallel_loop(0, n_q, 1, unroll=4)
        def _(a):
            for b in range(4):
                col = jnp.broadcast_to(4 * a + b, (_L,)).astype(jnp.int32)
                for h in range(2):
                    vals = plsc.load_gather(in_v.at[s],
                                            [rows16 + 16 * h, col])
                    out_v[s, a, pl.ds(32 * b + 16 * h, _L)] = vals

    def fire_out(c, s, n_q=32):
        pltpu.async_copy(out_v.at[s, pl.ds(0, n_q)],
                         tbl_q.at[pl.ds(c * 32, n_q)], oss[s])

    def wait_out(s):
        pltpu.make_async_copy(out_v.at[s], tbl_q.at[pl.ds(0, 32)],
                              oss[s]).wait()

    def blk(k):
        return k * _NW + wid

    fire_in(blk(0), 0)

    def pair(p, carry):
        fire_in(blk(2 * p + 1), 1)
        wait_in(0)

        @pl.when(p > 0)
        def _():
            wait_out(0)

        transpose(0)
        fire_out(blk(2 * p), 0)

        @pl.when(p < 121)
        def _():
            fire_in(blk(2 * p + 2), 0)

        wait_in(1)

        @pl.when(p > 0)
        def _():
            wait_out(1)

        transpose(1)
        fire_out(blk(2 * p + 1), 1)
        return carry

    lax.fori_loop(0, 122, pair, 0)
    wait_out(0)
    wait_out(1)

    @pl.when(wid < 4)
    def _():
        fire_in(7808 + wid, 0)
        wait_in(0)
        transpose(0)
        fire_out(7808 + wid, 0)
        wait_out(0)

    @pl.when(wid == 4)
    def _():
        pltpu.async_copy(emb_tail, tbl_q.at[pl.ds(7812 * 32, 16)], osa).wait()


def _gather_body(tbl_q, idx_t, out_t, idx_v, q_v, r_v, g_v, o_v,
                 isem, gsa, gsb, osa, osb):
    wid = _wid()
    rows16 = lax.iota(jnp.int32, _L)
    gss = (gsa, gsb)
    oss = (osa, osb)

    def prep(j, s):
        @plsc.parallel_loop(0, 8, 1, unroll=2)
        def _(b):
            v = idx_v[j, pl.ds(16 * b, _L)]
            q_v[s, pl.ds(16 * b, _L)] = v >> 2
            r_v[s, pl.ds(16 * b, _L)] = (v & 3) * 32

    def fire(s):
        pltpu.async_copy(tbl_q.at[q_v.at[s]], g_v.at[s], gss[s])

    def drain(s):
        pltpu.make_async_copy(tbl_q.at[pl.ds(0, 128)], g_v.at[s],
                              gss[s]).wait()

    def extract(s):
        bases = [r_v[s, pl.ds(16 * lb, _L)] for lb in range(8)]

        @plsc.parallel_loop(0, 32, 1, unroll=2)
        def _(f):
            for lb in range(8):
                rr = rows16 + 16 * lb
                vals = plsc.load_gather(g_v.at[s], [rr, bases[lb] + f])
                o_v[s, f, pl.ds(16 * lb, _L)] = vals

    def wb(j, c, s):
        pltpu.async_copy(o_v.at[s], out_t.at[j, :, pl.ds(c * 128, 128)],
                         oss[s])

    def wb_wait(s):
        pltpu.make_async_copy(o_v.at[s], out_t.at[0, :, pl.ds(0, 128)],
                              oss[s]).wait()

    def do_cblock(cc, carry):
        c = cc * _NW + wid
        pltpu.async_copy(idx_t.at[:, pl.ds(c * 128, 128)], idx_v, isem)
        pltpu.make_async_copy(idx_t.at[:, pl.ds(0, 128)], idx_v, isem).wait()
        prep(0, 0)
        fire(0)

        def pair(p, carry2):
            j0 = 2 * p
            prep(j0 + 1, 1)
            fire(1)
            drain(0)

            @pl.when((cc > 0) | (p > 0))
            def _():
                wb_wait(0)

            extract(0)
            wb(j0, c, 0)

            @pl.when(p < 24)
            def _():
                prep(j0 + 2, 0)
                fire(0)

            drain(1)

            @pl.when((cc > 0) | (p > 0))
            def _():
                wb_wait(1)

            extract(1)
            wb(j0 + 1, c, 1)
            return carry2

        lax.fori_loop(0, 25, pair, 0)
        return carry

    lax.fori_loop(0, 16384 // 128 // _NW, do_cblock, 0)
    wb_wait(0)
    wb_wait(1)


def kernel(inputs, embedding):
    if inputs.shape[-1] == 1:
        inputs = jnp.squeeze(inputs, axis=-1)
    n, m = inputs.shape
    dim = embedding.shape[1]

    emb_t = jnp.transpose(embedding)
    idx_t = jnp.transpose(inputs).astype(jnp.int32)
    n_tail = embedding.shape[0] % 512
    emb_tail = jnp.reshape(
        lax.slice(embedding, (embedding.shape[0] - n_tail, 0),
                  (embedding.shape[0], dim)), (n_tail * dim // 128, 128))
    mesh = plsc.VectorSubcoreMesh(core_axis_name="c", subcore_axis_name="s")
    params = pltpu.CompilerParams(use_tc_tiling_on_sc=True,
                                  needs_layout_passes=False)

    tbl_q = pl.kernel(
        _requad_body,
        out_type=jax.ShapeDtypeStruct((250000, 128), jnp.float32),
        mesh=mesh,
        compiler_params=params,
        scratch_types=[
            pltpu.VMEM((2, dim, 128), jnp.float32),
            pltpu.VMEM((2, 32, 128), jnp.float32),
            pltpu.SemaphoreType.DMA,
            pltpu.SemaphoreType.DMA,
            pltpu.SemaphoreType.DMA,
            pltpu.SemaphoreType.DMA,
        ],
    )(emb_t, emb_tail)

    out_t = pl.kernel(
        _gather_body,
        out_type=jax.ShapeDtypeStruct((m, dim, n), jnp.float32),
        mesh=mesh,
        compiler_params=params,
        scratch_types=[
            pltpu.VMEM((m, 128), jnp.int32),
            pltpu.VMEM((2, 128), jnp.int32),
            pltpu.VMEM((2, 128), jnp.int32),
            pltpu.VMEM((2, 128, 128), jnp.float32),
            pltpu.VMEM((2, dim, 128), jnp.float32),
            pltpu.SemaphoreType.DMA,
            pltpu.SemaphoreType.DMA,
            pltpu.SemaphoreType.DMA,
            pltpu.SemaphoreType.DMA,
            pltpu.SemaphoreType.DMA,
        ],
    )(tbl_q, idx_t)

    return jnp.transpose(out_t, (2, 0, 1))
